# Initial kernel scaffold; baseline (speedup 1.0000x reference)
import functools

import jax
import jax.numpy as jnp
from jax import lax
from jax.experimental import pallas as pl
from jax.experimental.pallas import tpu as pltpu
from jax.experimental.pallas import tpu_sc as plsc

N_SAMPLES = 3_200_000
N_RAYS = 100_000
NC = 2
NS = 16
NW = NC * NS
C_PER_W = N_SAMPLES // NW
CHUNK = 4_000
N_CHUNKS = C_PER_W // CHUNK
G = 80
N_G = CHUNK // G
ACC_ROWS = 100_352
STRIPE = ACC_ROWS // NS
LANES = 16


def _sc_segment_sum(normals, weights_flat, idx2d):
    @functools.partial(
        pl.kernel,
        out_type=jax.ShapeDtypeStruct((NC, ACC_ROWS, 4), jnp.float32),
        mesh=plsc.VectorSubcoreMesh(core_axis_name="c", subcore_axis_name="s"),
        scratch_types=[
            pltpu.MemorySpace.VMEM_SHARED((ACC_ROWS, 4), jnp.float32),
            pltpu.VMEM((STRIPE, 4), jnp.float32),
            pltpu.VMEM((CHUNK, 3), jnp.float32),
            pltpu.VMEM((CHUNK,), jnp.float32),
            pltpu.VMEM((N_G, G), jnp.int32),
        ],
    )
    def k(n_hbm, w_hbm, i_hbm, out_hbm, acc, vals, n_v, w_v, idx_v):
        cc = lax.axis_index("c")
        ss = lax.axis_index("s")
        wid = ss * NC + cc
        iota = lax.iota(jnp.int32, LANES)
        zeros = jnp.zeros((LANES,), jnp.float32)

        def zbody(p, carry):
            flat = iota + p * LANES
            plsc.store_scatter(vals, [flat >> 2, flat & 3], zeros)
            return carry

        lax.fori_loop(0, STRIPE * 4 // LANES, zbody, 0)
        pltpu.sync_copy(vals, acc.at[pl.ds(ss * STRIPE, STRIPE)])
        plsc.subcore_barrier()

        def chunk_body(kk, carry):
            base = wid * C_PER_W + kk * CHUNK
            pltpu.sync_copy(n_hbm.at[pl.ds(base, CHUNK)], n_v)
            pltpu.sync_copy(w_hbm.at[pl.ds(base, CHUNK)], w_v)
            pltpu.sync_copy(i_hbm.at[pl.ds(base // G, N_G)], idx_v)

            def jbody(j, c2):
                rows = iota + j * LANES
                wv = w_v[pl.ds(j * LANES, LANES)]
                for c in range(3):
                    cols = jnp.full((LANES,), c, jnp.int32)
                    nc_ = plsc.load_gather(n_v, [rows, cols])
                    plsc.store_scatter(vals, [rows, cols], wv * nc_)
                return c2

            lax.fori_loop(0, CHUNK // LANES, jbody, 0)

            def gbody(g, c3):
                pltpu.sync_copy(vals.at[pl.ds(g * G, G)],
                                acc.at[idx_v.at[g]], add=True)
                return c3

            lax.fori_loop(0, N_G, gbody, 0)
            return carry

        lax.fori_loop(0, N_CHUNKS, chunk_body, 0)

        plsc.subcore_barrier()
        pltpu.sync_copy(acc.at[pl.ds(ss * STRIPE, STRIPE)],
                        out_hbm.at[cc, pl.ds(ss * STRIPE, STRIPE)])

    return k(normals, weights_flat, idx2d)


def _merge_normalize(partial):
    BR = 512

    def body(x_ref, o_ref):
        x = x_ref[...]
        s = x[0] + x[1]
        sq = s * s
        nsq = sq[:, 0:1] + sq[:, 1:2] + sq[:, 2:3]
        o_ref[...] = s / jnp.sqrt(jnp.maximum(nsq, 1e-20))

    return pl.pallas_call(
        body,
        grid=(ACC_ROWS // BR,),
        in_specs=[pl.BlockSpec((NC, BR, 4), lambda i: (0, i, 0))],
        out_specs=pl.BlockSpec((BR, 4), lambda i: (i, 0)),
        out_shape=jax.ShapeDtypeStruct((ACC_ROWS, 4), jnp.float32),
    )(partial)


def kernel(normals, weights, ray_indices, num_rays):
    idx = ray_indices.astype(jnp.int32).reshape(N_SAMPLES // G, G)
    w = weights.reshape(N_SAMPLES)
    partial = _sc_segment_sum(normals, w, idx)
    merged = _merge_normalize(partial)
    return merged[:N_RAYS, :3]


# speedup vs baseline: 1.4121x; 1.4121x over previous
import functools

import jax
import jax.numpy as jnp
from jax import lax
from jax.experimental import pallas as pl
from jax.experimental.pallas import tpu as pltpu
from jax.experimental.pallas import tpu_sc as plsc

N_SAMPLES = 3_200_000
N_RAYS = 100_000
NC = 2
NS = 16
NW = NC * NS
C_PER_W = N_SAMPLES // NW
CHUNK = 4_000
N_CHUNKS = C_PER_W // CHUNK
G = 80
N_G = CHUNK // G
ACC_ROWS = 100_352
STRIPE = ACC_ROWS // NS
LANES = 16


def _sc_segment_sum(normals_flat, weights_flat, idx3d):
    @functools.partial(
        pl.kernel,
        out_type=jax.ShapeDtypeStruct((NC * 3, 1, ACC_ROWS), jnp.float32),
        mesh=plsc.VectorSubcoreMesh(core_axis_name="c", subcore_axis_name="s"),
        compiler_params=pltpu.CompilerParams(needs_layout_passes=False),
        scratch_types=[
            pltpu.MemorySpace.VMEM_SHARED((ACC_ROWS,), jnp.float32),
            pltpu.MemorySpace.VMEM_SHARED((ACC_ROWS,), jnp.float32),
            pltpu.MemorySpace.VMEM_SHARED((ACC_ROWS,), jnp.float32),
            pltpu.VMEM((STRIPE,), jnp.float32),
            pltpu.VMEM((3 * CHUNK,), jnp.float32),
            pltpu.VMEM((CHUNK,), jnp.float32),
            pltpu.VMEM((CHUNK,), jnp.float32),
            pltpu.VMEM((CHUNK,), jnp.float32),
            pltpu.VMEM((CHUNK,), jnp.float32),
            pltpu.VMEM((N_G, 1, G), jnp.int32),
        ],
    )
    def k(n_hbm, w_hbm, i_hbm, out_hbm,
          acc_x, acc_y, acc_z, zbuf, n_v, w_v, vx, vy, vz, idx_v):
        cc = lax.axis_index("c")
        ss = lax.axis_index("s")
        wid = ss * NC + cc
        iota = lax.iota(jnp.int32, LANES)
        zeros = jnp.zeros((LANES,), jnp.float32)
        accs = (acc_x, acc_y, acc_z)
        vplanes = (vx, vy, vz)

        def zbody(p, carry):
            zbuf[pl.ds(p * LANES, LANES)] = zeros
            return carry

        lax.fori_loop(0, STRIPE // LANES, zbody, 0)
        for a in accs:
            pltpu.sync_copy(zbuf, a.at[pl.ds(ss * STRIPE, STRIPE)])
        plsc.subcore_barrier()

        def chunk_body(kk, carry):
            base = wid * C_PER_W + kk * CHUNK
            pltpu.sync_copy(n_hbm.at[pl.ds(3 * base, 3 * CHUNK)], n_v)
            pltpu.sync_copy(w_hbm.at[pl.ds(base, CHUNK)], w_v)
            pltpu.sync_copy(i_hbm.at[pl.ds(base // G, N_G)], idx_v)

            def jbody(j, c2):
                off = j * LANES
                rows3 = (iota + off) * 3
                wv = w_v[pl.ds(off, LANES)]
                for c in range(3):
                    nc_ = plsc.load_gather(n_v, [rows3 + c])
                    vplanes[c][pl.ds(off, LANES)] = wv * nc_
                return c2

            lax.fori_loop(0, CHUNK // LANES, jbody, 0)

            def gbody(g, c3):
                for c in range(3):
                    pltpu.sync_copy(vplanes[c].at[pl.ds(g * G, G)],
                                    accs[c].at[idx_v.at[g, 0]], add=True)
                return c3

            lax.fori_loop(0, N_G, gbody, 0)
            return carry

        lax.fori_loop(0, N_CHUNKS, chunk_body, 0)

        plsc.subcore_barrier()
        for c in range(3):
            pltpu.sync_copy(
                accs[c].at[pl.ds(ss * STRIPE, STRIPE)],
                out_hbm.at[cc * 3 + c, 0, pl.ds(ss * STRIPE, STRIPE)])

    return k(normals_flat, weights_flat, idx3d)


def _merge_normalize(partial):
    BR = 512

    def body(x_ref, o_ref):
        x = x_ref[...]
        s = x[0] + x[1]
        nsq = jnp.sum(s * s, axis=0, keepdims=True)
        o_ref[...] = s / jnp.sqrt(jnp.maximum(nsq, 1e-20))

    return pl.pallas_call(
        body,
        grid=(ACC_ROWS // BR,),
        in_specs=[pl.BlockSpec((NC, 3, BR), lambda i: (0, 0, i))],
        out_specs=pl.BlockSpec((3, BR), lambda i: (0, i)),
        out_shape=jax.ShapeDtypeStruct((3, ACC_ROWS), jnp.float32),
    )(partial)


def kernel(normals, weights, ray_indices, num_rays):
    idx = ray_indices.astype(jnp.int32).reshape(N_SAMPLES // G, 1, G)
    w = weights.reshape(N_SAMPLES)
    n = normals.reshape(3 * N_SAMPLES)
    partial = _sc_segment_sum(n, w, idx)
    merged = _merge_normalize(partial.reshape(NC, 3, ACC_ROWS))
    return merged[:, :N_RAYS].T


# speedup vs baseline: 1.4428x; 1.0217x over previous
import functools

import jax
import jax.numpy as jnp
from jax import lax
from jax.experimental import pallas as pl
from jax.experimental.pallas import tpu as pltpu
from jax.experimental.pallas import tpu_sc as plsc

N_SAMPLES = 3_200_000
N_RAYS = 100_000
NC = 2
NS = 16
NW = NC * NS
C_PER_W = N_SAMPLES // NW
CHUNK = 4_000
N_CHUNKS = C_PER_W // CHUNK
G = 80
N_G = CHUNK // G
ACC_ROWS = 100_352
STRIPE = ACC_ROWS // NS
LANES = 16


def _sc_segment_sum(normals_flat, weights_flat, idx3d):
    @functools.partial(
        pl.kernel,
        out_type=jax.ShapeDtypeStruct((NC * 3, 1, ACC_ROWS), jnp.float32),
        mesh=plsc.VectorSubcoreMesh(core_axis_name="c", subcore_axis_name="s"),
        compiler_params=pltpu.CompilerParams(needs_layout_passes=False),
        scratch_types=[
            pltpu.MemorySpace.VMEM_SHARED((ACC_ROWS,), jnp.float32),
            pltpu.MemorySpace.VMEM_SHARED((ACC_ROWS,), jnp.float32),
            pltpu.MemorySpace.VMEM_SHARED((ACC_ROWS,), jnp.float32),
            pltpu.VMEM((STRIPE,), jnp.float32),
            pltpu.VMEM((2 * 3 * CHUNK,), jnp.float32),
            pltpu.VMEM((2 * CHUNK,), jnp.float32),
            pltpu.VMEM((2 * CHUNK,), jnp.float32),
            pltpu.VMEM((2 * CHUNK,), jnp.float32),
            pltpu.VMEM((2 * CHUNK,), jnp.float32),
            pltpu.VMEM((2, N_G, 1, G), jnp.int32),
            pltpu.SemaphoreType.DMA,
            pltpu.SemaphoreType.DMA,
        ],
    )
    def k(n_hbm, w_hbm, i_hbm, out_hbm,
          acc_x, acc_y, acc_z, zbuf, n_v, w_v, vx, vy, vz, idx_v,
          sem_in, sem_s):
        cc = lax.axis_index("c")
        ss = lax.axis_index("s")
        wid = ss * NC + cc
        iota = lax.iota(jnp.int32, LANES)
        zeros = jnp.zeros((LANES,), jnp.float32)
        accs = (acc_x, acc_y, acc_z)
        vplanes = (vx, vy, vz)

        def in_copies(kk, b):
            base = wid * C_PER_W + kk * CHUNK
            return (
                pltpu.make_async_copy(
                    n_hbm.at[pl.ds(3 * base, 3 * CHUNK)],
                    n_v.at[pl.ds(b * 3 * CHUNK, 3 * CHUNK)], sem_in),
                pltpu.make_async_copy(
                    w_hbm.at[pl.ds(base, CHUNK)],
                    w_v.at[pl.ds(b * CHUNK, CHUNK)], sem_in),
                pltpu.make_async_copy(
                    i_hbm.at[pl.ds(base // G, N_G)], idx_v.at[b], sem_in),
            )

        def scat_copies(g, b):
            return tuple(
                pltpu.make_async_copy(
                    vplanes[c].at[pl.ds(b * CHUNK + g * G, G)],
                    accs[c].at[idx_v.at[b, g, 0]], sem_s)
                for c in range(3))

        def zbody(p, carry):
            zbuf[pl.ds(p * LANES, LANES)] = zeros
            return carry

        lax.fori_loop(0, STRIPE // LANES, zbody, 0)
        for a in accs:
            pltpu.sync_copy(zbuf, a.at[pl.ds(ss * STRIPE, STRIPE)])
        plsc.subcore_barrier()

        for cp in in_copies(0, 0):
            cp.start()

        def chunk_body(kk, carry):
            b = lax.rem(kk, 2)
            for cp in in_copies(kk, b):
                cp.wait()

            def jbody(j, c2):
                off = j * LANES
                rows3 = (b * 3 * CHUNK) + (iota + off) * 3
                wv = w_v[pl.ds(b * CHUNK + off, LANES)]
                for c in range(3):
                    nc_ = plsc.load_gather(n_v, [rows3 + c])
                    vplanes[c][pl.ds(b * CHUNK + off, LANES)] = wv * nc_
                return c2

            lax.fori_loop(0, CHUNK // LANES, jbody, 0)

            @pl.when(kk > 0)
            def _():
                def dbody(g, c4):
                    for cp in scat_copies(g, 1 - b):
                        cp.wait()
                    return c4
                lax.fori_loop(0, N_G, dbody, 0)

            @pl.when(kk + 1 < N_CHUNKS)
            def _():
                for cp in in_copies(kk + 1, 1 - b):
                    cp.start()

            def gbody(g, c3):
                for cp in scat_copies(g, b):
                    cp.start(add=True)
                return c3

            lax.fori_loop(0, N_G, gbody, 0)
            return carry

        lax.fori_loop(0, N_CHUNKS, chunk_body, 0)

        def dlast(g, c5):
            for cp in scat_copies(g, lax.rem(N_CHUNKS - 1, 2)):
                cp.wait()
            return c5

        lax.fori_loop(0, N_G, dlast, 0)

        plsc.subcore_barrier()
        for c in range(3):
            pltpu.sync_copy(
                accs[c].at[pl.ds(ss * STRIPE, STRIPE)],
                out_hbm.at[cc * 3 + c, 0, pl.ds(ss * STRIPE, STRIPE)])

    return k(normals_flat, weights_flat, idx3d)


def _merge_normalize(partial):
    BR = 512

    def body(x_ref, o_ref):
        x = x_ref[...]
        s = x[0] + x[1]
        nsq = jnp.sum(s * s, axis=0, keepdims=True)
        o_ref[...] = s / jnp.sqrt(jnp.maximum(nsq, 1e-20))

    return pl.pallas_call(
        body,
        grid=(ACC_ROWS // BR,),
        in_specs=[pl.BlockSpec((NC, 3, BR), lambda i: (0, 0, i))],
        out_specs=pl.BlockSpec((3, BR), lambda i: (0, i)),
        out_shape=jax.ShapeDtypeStruct((3, ACC_ROWS), jnp.float32),
    )(partial)


def kernel(normals, weights, ray_indices, num_rays):
    idx = ray_indices.astype(jnp.int32).reshape(N_SAMPLES // G, 1, G)
    w = weights.reshape(N_SAMPLES)
    n = normals.reshape(3 * N_SAMPLES)
    partial = _sc_segment_sum(n, w, idx)
    merged = _merge_normalize(partial.reshape(NC, 3, ACC_ROWS))
    return merged[:, :N_RAYS].T
